# baseline (speedup 1.0000x reference)
import functools

import jax
import jax.numpy as jnp
import numpy as np
from jax.experimental import pallas as pl
from jax.experimental.pallas import tpu as pltpu

_D_MODEL = 256
_NUM_HEADS = 4
_HEAD_DIM = _D_MODEL // _NUM_HEADS
_TAU = 1.0
_GAMMA = 0.3
_BETA = 1.0
_ETA = 1.0
_NSEG = 8
_QB = 256
_KB = 256
_MINIT = -1e38
_QK_SCALE = _BETA / np.sqrt(_HEAD_DIM) / _TAU
_SIG_SCALE = 2.0 * _GAMMA / _TAU


def _fused_body(qp_ref, kp_ref,
                phi_q_ref, sq_ref,
                phi_k_ref, sk_ref, szk_ref,
                wa_ref, wb_ref, wv_ref,
                out_ref,
                pk_s, pv_s, ck_s):
    i = pl.program_id(0)
    qs = i * _QB
    f32 = jnp.float32
    dn_t = (((1,), (1,)), ((), ()))
    dn_m = (((1,), (0,)), ((), ()))
    nk = phi_k_ref.shape[0]

    @pl.when(i == 0)
    def _project_keys():
        def kinit(b, _):
            koff = b * _KB
            phik = phi_k_ref[pl.ds(koff, _KB), :]
            pk_s[pl.ds(koff, _KB), :] = jax.lax.dot_general(
                phik, wb_ref[...], dn_t,
                preferred_element_type=f32).astype(jnp.bfloat16)
            pv_s[pl.ds(koff, _KB), :] = jax.lax.dot_general(
                phik, wv_ref[...], dn_t,
                preferred_element_type=f32).astype(jnp.bfloat16)
            sk = sk_ref[pl.ds(koff, _KB), :]
            ones_row = jnp.ones((1, sk.shape[1]), f32)
            kn = jax.lax.dot_general(ones_row, sk * sk, dn_t,
                                     preferred_element_type=f32)
            ck_s[:, pl.ds(koff, _KB)] = (
                -_GAMMA * kn + _ETA * jnp.log1p(szk_ref[:, pl.ds(koff, _KB)])
            ) / _TAU
            return 0
        jax.lax.fori_loop(0, nk // _KB, kinit, 0)

    pq = (jax.lax.dot_general(phi_q_ref[...], wa_ref[...], dn_t,
                              preferred_element_type=f32)
          * _QK_SCALE).astype(jnp.bfloat16)

    s0 = jnp.int32(0)
    s1 = jnp.int32(0)
    for j in range(1, _NSEG):
        s0 = s0 + (qp_ref[j] <= qs).astype(jnp.int32)
        s1 = s1 + (qp_ref[j] <= qs + _QB - 1).astype(jnp.int32)
    k_lo = kp_ref[s0]
    k_hi = kp_ref[s1 + 1]
    blo = k_lo // _KB
    bhi = (k_hi + _KB - 1) // _KB

    rows = qs + jax.lax.broadcasted_iota(jnp.int32, (_QB, 1), 0)
    seg_q = jnp.zeros((_QB, 1), jnp.int32)
    for j in range(1, _NSEG):
        seg_q = seg_q + (qp_ref[j] <= rows).astype(jnp.int32)

    sq = sq_ref[...]

    def body(b, carry):
        ms, ls, accs = carry
        koff = b * _KB
        pk = pk_s[pl.ds(koff, _KB), :]
        pv = pv_s[pl.ds(koff, _KB), :]
        sk = sk_ref[pl.ds(koff, _KB), :]
        ck = ck_s[:, pl.ds(koff, _KB)]

        sigdot = jax.lax.dot_general(sq, sk, dn_t,
                                     preferred_element_type=f32)
        common = _SIG_SCALE * sigdot + ck

        cols = koff + jax.lax.broadcasted_iota(jnp.int32, (1, _KB), 1)
        seg_k = jnp.zeros((1, _KB), jnp.int32)
        for j in range(1, _NSEG):
            seg_k = seg_k + (kp_ref[j] <= cols).astype(jnp.int32)
        negmask = jnp.where(seg_q == seg_k, 0.0, -jnp.inf)
        common = common + negmask

        new_ms, new_ls, new_accs = [], [], []
        for h in range(_NUM_HEADS):
            sl = slice(h * _HEAD_DIM, (h + 1) * _HEAD_DIM)
            s = common + jax.lax.dot_general(
                pq[:, sl], pk[:, sl], dn_t, preferred_element_type=f32)
            m_new = jnp.maximum(ms[h], jnp.max(s, axis=1, keepdims=True))
            p = jnp.exp(s - m_new)
            alpha = jnp.exp(ms[h] - m_new)
            new_ms.append(m_new)
            new_ls.append(ls[h] * alpha + jnp.sum(p, axis=1, keepdims=True))
            new_accs.append(accs[h] * alpha + jax.lax.dot_general(
                p.astype(jnp.bfloat16), pv[:, sl], dn_m,
                preferred_element_type=f32))
        return tuple(new_ms), tuple(new_ls), tuple(new_accs)

    m0 = tuple(jnp.full((_QB, 1), _MINIT, f32) for _ in range(_NUM_HEADS))
    l0 = tuple(jnp.zeros((_QB, 1), f32) for _ in range(_NUM_HEADS))
    a0 = tuple(jnp.zeros((_QB, _HEAD_DIM), f32) for _ in range(_NUM_HEADS))
    ms, ls, accs = jax.lax.fori_loop(blo, bhi, body, (m0, l0, a0))

    for h in range(_NUM_HEADS):
        sl = slice(h * _HEAD_DIM, (h + 1) * _HEAD_DIM)
        out_ref[:, sl] = accs[h] / jnp.maximum(ls[h], 1e-20)


@functools.partial(jax.jit, static_argnames=("interpret",))
def _run(phi_q, sig_q, q_ptrs, phi_k, sig_k, size_k, k_ptrs, W_A, W_B, W_V,
         interpret=False):
    nq, d = phi_q.shape
    nk = phi_k.shape[0]
    dsig = sig_q.shape[1]
    nqb = nq // _QB
    szk2d = size_k.reshape(1, nk)

    grid_spec = pltpu.PrefetchScalarGridSpec(
        num_scalar_prefetch=2,
        grid=(nqb,),
        in_specs=[
            pl.BlockSpec((_QB, d), lambda i, qp, kp: (i, 0)),
            pl.BlockSpec((_QB, dsig), lambda i, qp, kp: (i, 0)),
            pl.BlockSpec((nk, d), lambda i, qp, kp: (0, 0)),
            pl.BlockSpec((nk, dsig), lambda i, qp, kp: (0, 0)),
            pl.BlockSpec((1, nk), lambda i, qp, kp: (0, 0)),
            pl.BlockSpec((d, d), lambda i, qp, kp: (0, 0)),
            pl.BlockSpec((d, d), lambda i, qp, kp: (0, 0)),
            pl.BlockSpec((d, d), lambda i, qp, kp: (0, 0)),
        ],
        out_specs=pl.BlockSpec((_QB, d), lambda i, qp, kp: (i, 0)),
        scratch_shapes=[
            pltpu.VMEM((nk, d), jnp.bfloat16),
            pltpu.VMEM((nk, d), jnp.bfloat16),
            pltpu.VMEM((1, nk), jnp.float32),
        ],
    )
    out = pl.pallas_call(
        _fused_body,
        grid_spec=grid_spec,
        out_shape=jax.ShapeDtypeStruct((nq, d), jnp.float32),
        compiler_params=pltpu.CompilerParams(
            dimension_semantics=("arbitrary",)),
        interpret=interpret,
    )(q_ptrs, k_ptrs, phi_q, sig_q, phi_k, sig_k, szk2d, W_A, W_B, W_V)
    return out


def kernel(phi_q, sig_q, size_q, q_ptrs, phi_k, sig_k, size_k, k_ptrs,
           W_A, W_B, W_V):
    out = _run(phi_q, sig_q, q_ptrs, phi_k, sig_k, size_k, k_ptrs,
               W_A, W_B, W_V)
    nq = phi_q.shape[0]
    return (out.reshape(nq, _NUM_HEADS, _HEAD_DIM), q_ptrs)
